# baseline (speedup 1.0000x reference)
import functools

import jax
import jax.numpy as jnp
from jax.experimental import pallas as pl
from jax.experimental.pallas import tpu as pltpu

_K = 32
_R2 = 0.25 * 0.25


def _tile_kernel(xyz_row_ref, xyzT_ref, table_ref, pts_row_ref,
                 W1_ref, b1_ref, W2_ref, b2_ref,
                 Wsc_ref, bsc_ref, Wm1_ref, bm1_ref, Wm2_ref, bm2_ref,
                 out_ref, D_ref, *, R, N, FEAT):
    xyz_row = xyz_row_ref[0]
    xyzT = xyzT_ref[0]
    table = table_ref[0]
    pts_row = pts_row_ref[0]

    sq_all = jnp.sum(xyzT * xyzT, axis=0, keepdims=True)
    sq_row = jnp.sum(xyz_row * xyz_row, axis=1, keepdims=True)
    cross = jnp.dot(xyz_row, xyzT, preferred_element_type=jnp.float32)
    D0 = sq_row + sq_all - 2.0 * cross

    W1 = W1_ref[...]
    b1 = b1_ref[...]
    W2 = W2_ref[...]
    b2 = b2_ref[...]

    D_ref[...] = D0
    m0 = jnp.min(D0, axis=1, keepdims=True)

    def step(_, carry):
        m, acc = carry
        D = D_ref[...]
        sel = D == m
        onehot = jnp.where(sel, 1.0, 0.0).astype(jnp.bfloat16)
        live = D > m
        m_next = jnp.min(jnp.where(live, D, jnp.inf), axis=1, keepdims=True)
        feat = jnp.dot(onehot, table, preferred_element_type=jnp.float32)
        knn_xyz = feat[:, 0:3]
        knn_pts = feat[:, 3:3 + FEAT]
        dist = m
        far = dist > _R2
        knn_xyz = jnp.where(far, xyz_row, knn_xyz)
        knn_pts = jnp.where(far, pts_row, knn_pts)
        dist = jnp.where(far, 0.0, dist)
        rel = knn_xyz - xyz_row
        g7 = jnp.concatenate([knn_xyz, rel, dist], axis=1)
        h = jnp.maximum(jnp.dot(g7, W1, preferred_element_type=jnp.float32) + b1, 0.0)
        kern = jnp.dot(h, W2, preferred_element_type=jnp.float32) + b2
        gpoints = jnp.concatenate([g7, knn_pts], axis=1)
        acc = acc + kern * gpoints
        return m_next, acc

    acc0 = jnp.zeros((R, 7 + FEAT), dtype=jnp.float32)
    _, acc = jax.lax.fori_loop(0, _K, step, (m0, acc0))
    gp = acc * (1.0 / _K)
    gp67 = jnp.concatenate([xyz_row, gp], axis=1)
    g1 = jnp.maximum(
        jnp.dot(gp67, Wm1_ref[...], preferred_element_type=jnp.float32) + bm1_ref[...], 0.0)
    g2 = jnp.dot(g1, Wm2_ref[...], preferred_element_type=jnp.float32) + bm2_ref[...]
    res = jnp.dot(pts_row, Wsc_ref[...], preferred_element_type=jnp.float32) + bsc_ref[...]
    out_ref[0] = jnp.maximum(res + g2, 0.0)


def kernel(xyz, points, W_dfg1, b_dfg1, W_dfg2, b_dfg2, W_sc, b_sc,
           W_m1, b_m1, W_m2, b_m2):
    B, N, _ = xyz.shape
    FEAT = points.shape[-1]
    OUT = W_m2.shape[-1]
    R = 256 if N % 256 == 0 else N

    xyzT = jnp.transpose(xyz, (0, 2, 1))
    table = jnp.concatenate([xyz, points], axis=-1).astype(jnp.bfloat16)

    grid = (B, N // R)
    body = functools.partial(_tile_kernel, R=R, N=N, FEAT=FEAT)

    out = pl.pallas_call(
        body,
        grid=grid,
        in_specs=[
            pl.BlockSpec((1, R, 3), lambda b, r: (b, r, 0)),
            pl.BlockSpec((1, 3, N), lambda b, r: (b, 0, 0)),
            pl.BlockSpec((1, N, 3 + FEAT), lambda b, r: (b, 0, 0)),
            pl.BlockSpec((1, R, FEAT), lambda b, r: (b, r, 0)),
            pl.BlockSpec(W_dfg1.shape, lambda b, r: (0, 0)),
            pl.BlockSpec((1, b_dfg1.shape[0]), lambda b, r: (0, 0)),
            pl.BlockSpec(W_dfg2.shape, lambda b, r: (0, 0)),
            pl.BlockSpec((1, b_dfg2.shape[0]), lambda b, r: (0, 0)),
            pl.BlockSpec(W_sc.shape, lambda b, r: (0, 0)),
            pl.BlockSpec((1, b_sc.shape[0]), lambda b, r: (0, 0)),
            pl.BlockSpec(W_m1.shape, lambda b, r: (0, 0)),
            pl.BlockSpec((1, b_m1.shape[0]), lambda b, r: (0, 0)),
            pl.BlockSpec(W_m2.shape, lambda b, r: (0, 0)),
            pl.BlockSpec((1, b_m2.shape[0]), lambda b, r: (0, 0)),
        ],
        out_specs=pl.BlockSpec((1, R, OUT), lambda b, r: (b, r, 0)),
        out_shape=jax.ShapeDtypeStruct((B, N, OUT), jnp.float32),
        scratch_shapes=[pltpu.VMEM((R, N), jnp.float32)],
        compiler_params=pltpu.CompilerParams(
            dimension_semantics=("parallel", "parallel"),
        ),
    )(xyz, xyzT, table, points,
      W_dfg1, b_dfg1[None, :], W_dfg2, b_dfg2[None, :],
      W_sc, b_sc[None, :], W_m1, b_m1[None, :], W_m2, b_m2[None, :])
    return out


# speedup vs baseline: 1.2857x; 1.2857x over previous
import functools

import jax
import jax.numpy as jnp
from jax.experimental import pallas as pl
from jax.experimental.pallas import tpu as pltpu

_K = 32
_R2 = 0.25 * 0.25


def _tile_kernel(xyz_row_ref, xyzT_ref, table_ref, pts_row_ref,
                 W1_ref, b1_ref, W2_ref, b2_ref,
                 Wsc_ref, bsc_ref, Wm1_ref, bm1_ref, Wm2_ref, bm2_ref,
                 out_ref, D_ref, *, R, N, FEAT):
    xyz_row = xyz_row_ref[0]
    xyzT = xyzT_ref[0]
    table = table_ref[0]
    pts_row = pts_row_ref[0]

    sq_all = jnp.sum(xyzT * xyzT, axis=0, keepdims=True)
    sq_row = jnp.sum(xyz_row * xyz_row, axis=1, keepdims=True)
    cross = jnp.dot(xyz_row, xyzT, preferred_element_type=jnp.float32)
    D0 = sq_row + sq_all - 2.0 * cross

    W1 = W1_ref[...]
    b1 = b1_ref[...]
    W2 = W2_ref[...]
    b2 = b2_ref[...]

    D_ref[...] = D0
    m0 = jnp.min(D0, axis=1, keepdims=True)

    def step(_, carry):
        m, acc = carry
        D = D_ref[...]
        sel = D == m
        onehot = sel.astype(jnp.float32)
        live = D > m
        m_next = jnp.min(jnp.where(live, D, jnp.inf), axis=1, keepdims=True)
        feat = jnp.dot(onehot, table, preferred_element_type=jnp.float32)
        knn_xyz = feat[:, 0:3]
        knn_pts = feat[:, 3:3 + FEAT]
        dist = m
        far = dist > _R2
        knn_xyz = jnp.where(far, xyz_row, knn_xyz)
        knn_pts = jnp.where(far, pts_row, knn_pts)
        dist = jnp.where(far, 0.0, dist)
        rel = knn_xyz - xyz_row
        g7 = jnp.concatenate([knn_xyz, rel, dist], axis=1)
        h = jnp.maximum(jnp.dot(g7, W1, preferred_element_type=jnp.float32) + b1, 0.0)
        kern = jnp.dot(h, W2, preferred_element_type=jnp.float32) + b2
        gpoints = jnp.concatenate([g7, knn_pts], axis=1)
        acc = acc + kern * gpoints
        return m_next, acc

    acc0 = jnp.zeros((R, 7 + FEAT), dtype=jnp.float32)
    _, acc = jax.lax.fori_loop(0, _K, step, (m0, acc0))
    gp = acc * (1.0 / _K)
    gp67 = jnp.concatenate([xyz_row, gp], axis=1)
    g1 = jnp.maximum(
        jnp.dot(gp67, Wm1_ref[...], preferred_element_type=jnp.float32) + bm1_ref[...], 0.0)
    g2 = jnp.dot(g1, Wm2_ref[...], preferred_element_type=jnp.float32) + bm2_ref[...]
    res = jnp.dot(pts_row, Wsc_ref[...], preferred_element_type=jnp.float32) + bsc_ref[...]
    out_ref[0] = jnp.maximum(res + g2, 0.0)


def kernel(xyz, points, W_dfg1, b_dfg1, W_dfg2, b_dfg2, W_sc, b_sc,
           W_m1, b_m1, W_m2, b_m2):
    B, N, _ = xyz.shape
    FEAT = points.shape[-1]
    OUT = W_m2.shape[-1]
    R = 512 if N % 512 == 0 else N

    xyzT = jnp.transpose(xyz, (0, 2, 1))
    table = jnp.concatenate([xyz, points], axis=-1)

    grid = (B, N // R)
    body = functools.partial(_tile_kernel, R=R, N=N, FEAT=FEAT)

    out = pl.pallas_call(
        body,
        grid=grid,
        in_specs=[
            pl.BlockSpec((1, R, 3), lambda b, r: (b, r, 0)),
            pl.BlockSpec((1, 3, N), lambda b, r: (b, 0, 0)),
            pl.BlockSpec((1, N, 3 + FEAT), lambda b, r: (b, 0, 0)),
            pl.BlockSpec((1, R, FEAT), lambda b, r: (b, r, 0)),
            pl.BlockSpec(W_dfg1.shape, lambda b, r: (0, 0)),
            pl.BlockSpec((1, b_dfg1.shape[0]), lambda b, r: (0, 0)),
            pl.BlockSpec(W_dfg2.shape, lambda b, r: (0, 0)),
            pl.BlockSpec((1, b_dfg2.shape[0]), lambda b, r: (0, 0)),
            pl.BlockSpec(W_sc.shape, lambda b, r: (0, 0)),
            pl.BlockSpec((1, b_sc.shape[0]), lambda b, r: (0, 0)),
            pl.BlockSpec(W_m1.shape, lambda b, r: (0, 0)),
            pl.BlockSpec((1, b_m1.shape[0]), lambda b, r: (0, 0)),
            pl.BlockSpec(W_m2.shape, lambda b, r: (0, 0)),
            pl.BlockSpec((1, b_m2.shape[0]), lambda b, r: (0, 0)),
        ],
        out_specs=pl.BlockSpec((1, R, OUT), lambda b, r: (b, r, 0)),
        out_shape=jax.ShapeDtypeStruct((B, N, OUT), jnp.float32),
        scratch_shapes=[pltpu.VMEM((R, N), jnp.float32)],
        compiler_params=pltpu.CompilerParams(
            dimension_semantics=("parallel", "parallel"),
        ),
    )(xyz, xyzT, table, points,
      W_dfg1, b_dfg1[None, :], W_dfg2, b_dfg2[None, :],
      W_sc, b_sc[None, :], W_m1, b_m1[None, :], W_m2, b_m2[None, :])
    return out


# speedup vs baseline: 1.4448x; 1.1237x over previous
import functools

import jax
import jax.numpy as jnp
from jax.experimental import pallas as pl
from jax.experimental.pallas import tpu as pltpu

_K = 32
_R2 = 0.25 * 0.25


def _tile_kernel(xyz_row_ref, xyzT_ref, table_ref, pts_row_ref,
                 W1_ref, b1_ref, W2_ref, b2_ref,
                 Wsc_ref, bsc_ref, Wm1_ref, bm1_ref, Wm2_ref, bm2_ref,
                 out_ref, D_ref, *, R, N, FEAT):
    xyz_row = xyz_row_ref[0]
    xyzT = xyzT_ref[0]
    table = table_ref[0]
    pts_row = pts_row_ref[0]

    sq_all = jnp.sum(xyzT * xyzT, axis=0, keepdims=True)
    sq_row = jnp.sum(xyz_row * xyz_row, axis=1, keepdims=True)
    cross = jnp.dot(xyz_row, xyzT, preferred_element_type=jnp.float32)
    D0 = sq_row + sq_all - 2.0 * cross

    W1 = W1_ref[...]
    b1 = b1_ref[...]
    W2 = W2_ref[...]
    b2 = b2_ref[...]

    D_ref[...] = D0
    m0 = jnp.min(D0, axis=1, keepdims=True)

    def step(_, carry):
        m, acc = carry
        D = D_ref[...]
        sel = D == m
        onehot = sel.astype(jnp.float32)
        live = D > m
        m_next = jnp.min(jnp.where(live, D, jnp.inf), axis=1, keepdims=True)
        feat = jnp.dot(onehot, table, preferred_element_type=jnp.float32)
        knn_xyz = feat[:, 0:3]
        knn_pts = feat[:, 3:3 + FEAT]
        dist = m
        far = dist > _R2
        knn_xyz = jnp.where(far, xyz_row, knn_xyz)
        knn_pts = jnp.where(far, pts_row, knn_pts)
        dist = jnp.where(far, 0.0, dist)
        rel = knn_xyz - xyz_row
        g7 = jnp.concatenate([knn_xyz, rel, dist], axis=1)
        h = jnp.maximum(jnp.dot(g7, W1, preferred_element_type=jnp.float32) + b1, 0.0)
        kern = jnp.dot(h, W2, preferred_element_type=jnp.float32) + b2
        gpoints = jnp.concatenate([g7, knn_pts], axis=1)
        acc = acc + kern * gpoints
        return m_next, acc

    acc0 = jnp.zeros((R, 7 + FEAT), dtype=jnp.float32)
    _, acc = jax.lax.fori_loop(0, _K, step, (m0, acc0))
    gp = acc * (1.0 / _K)
    gp67 = jnp.concatenate([xyz_row, gp], axis=1)
    g1 = jnp.maximum(
        jnp.dot(gp67, Wm1_ref[...], preferred_element_type=jnp.float32) + bm1_ref[...], 0.0)
    g2 = jnp.dot(g1, Wm2_ref[...], preferred_element_type=jnp.float32) + bm2_ref[...]
    res = jnp.dot(pts_row, Wsc_ref[...], preferred_element_type=jnp.float32) + bsc_ref[...]
    out_ref[0] = jnp.maximum(res + g2, 0.0)


def kernel(xyz, points, W_dfg1, b_dfg1, W_dfg2, b_dfg2, W_sc, b_sc,
           W_m1, b_m1, W_m2, b_m2):
    B, N, _ = xyz.shape
    FEAT = points.shape[-1]
    OUT = W_m2.shape[-1]
    R = 1024 if N % 1024 == 0 else N

    xyzT = jnp.transpose(xyz, (0, 2, 1))
    table = jnp.concatenate([xyz, points], axis=-1)

    grid = (B, N // R)
    body = functools.partial(_tile_kernel, R=R, N=N, FEAT=FEAT)

    out = pl.pallas_call(
        body,
        grid=grid,
        in_specs=[
            pl.BlockSpec((1, R, 3), lambda b, r: (b, r, 0)),
            pl.BlockSpec((1, 3, N), lambda b, r: (b, 0, 0)),
            pl.BlockSpec((1, N, 3 + FEAT), lambda b, r: (b, 0, 0)),
            pl.BlockSpec((1, R, FEAT), lambda b, r: (b, r, 0)),
            pl.BlockSpec(W_dfg1.shape, lambda b, r: (0, 0)),
            pl.BlockSpec((1, b_dfg1.shape[0]), lambda b, r: (0, 0)),
            pl.BlockSpec(W_dfg2.shape, lambda b, r: (0, 0)),
            pl.BlockSpec((1, b_dfg2.shape[0]), lambda b, r: (0, 0)),
            pl.BlockSpec(W_sc.shape, lambda b, r: (0, 0)),
            pl.BlockSpec((1, b_sc.shape[0]), lambda b, r: (0, 0)),
            pl.BlockSpec(W_m1.shape, lambda b, r: (0, 0)),
            pl.BlockSpec((1, b_m1.shape[0]), lambda b, r: (0, 0)),
            pl.BlockSpec(W_m2.shape, lambda b, r: (0, 0)),
            pl.BlockSpec((1, b_m2.shape[0]), lambda b, r: (0, 0)),
        ],
        out_specs=pl.BlockSpec((1, R, OUT), lambda b, r: (b, r, 0)),
        out_shape=jax.ShapeDtypeStruct((B, N, OUT), jnp.float32),
        scratch_shapes=[pltpu.VMEM((R, N), jnp.float32)],
        compiler_params=pltpu.CompilerParams(
            dimension_semantics=("parallel", "parallel"),
        ),
    )(xyz, xyzT, table, points,
      W_dfg1, b_dfg1[None, :], W_dfg2, b_dfg2[None, :],
      W_sc, b_sc[None, :], W_m1, b_m1[None, :], W_m2, b_m2[None, :])
    return out
